# baseline (speedup 1.0000x reference)
import jax
import jax.numpy as jnp
from jax.experimental import pallas as pl
from jax.experimental.pallas import tpu as pltpu

N_BOXES = 20000
B = 256
T = 80
P = T * B
NMS_THRESH = 0.5


def _nms_tiles_kernel(xr_ref, keep_ref, ident_ref, pk_ref):
    ident_ref[...] = (
        jax.lax.broadcasted_iota(jnp.int32, (B, B), 0)
        == jax.lax.broadcasted_iota(jnp.int32, (B, B), 1)
    ).astype(jnp.float32)
    pk_ref[...] = jnp.zeros((4, T + 1, 1, B), jnp.float32)

    tri = (
        jax.lax.broadcasted_iota(jnp.int32, (B, B), 0)
        < jax.lax.broadcasted_iota(jnp.int32, (B, B), 1)
    )
    ltf = jnp.where(
        jax.lax.broadcasted_iota(jnp.int32, (B, B), 1)
        < jax.lax.broadcasted_iota(jnp.int32, (B, B), 0), 1.0, 0.0)
    lane = jax.lax.broadcasted_iota(jnp.int32, (1, B), 1).astype(jnp.float32)

    def trans(row):
        return jnp.sum(ident_ref[...] * row, axis=1, keepdims=True)

    def trans_back(col):
        return jnp.sum(ident_ref[...] * col, axis=0, keepdims=True)

    def tile_body(t, m):
        rx1 = xr_ref[0, t]
        ry1 = xr_ref[1, t]
        rx2 = xr_ref[2, t]
        ry2 = xr_ref[3, t]
        cx1 = trans(rx1)
        cy1 = trans(ry1)
        cx2 = trans(rx2)
        cy2 = trans(ry2)
        ra = (rx2 - rx1) * (ry2 - ry1)
        ca = (cx2 - cx1) * (cy2 - cy1)

        def sweep(p, sup):
            px1 = pk_ref[0, p]
            py1 = pk_ref[1, p]
            px2 = pk_ref[2, p]
            py2 = pk_ref[3, p]
            pa = (px2 - px1) * (py2 - py1)
            xx1 = jnp.maximum(px1, cx1)
            yy1 = jnp.maximum(py1, cy1)
            xx2 = jnp.minimum(px2, cx2)
            yy2 = jnp.minimum(py2, cy2)
            w = jnp.maximum(1e-10, xx2 - xx1)
            h = jnp.maximum(1e-10, yy2 - yy1)
            inter = w * h
            iou = inter / (pa + ca - inter + 1e-14)
            s = jnp.max(jnp.where(iou > NMS_THRESH, 1.0, 0.0),
                        axis=1, keepdims=True)
            return jnp.maximum(sup, s)

        n_blocks = (m + (B - 1)) // B
        sup_col = jax.lax.fori_loop(0, n_blocks, sweep,
                                    jnp.zeros((B, 1), jnp.float32))
        a_row = trans_back(1.0 - sup_col)

        xx1 = jnp.maximum(cx1, rx1)
        yy1 = jnp.maximum(cy1, ry1)
        xx2 = jnp.minimum(cx2, rx2)
        yy2 = jnp.minimum(cy2, ry2)
        w = jnp.maximum(1e-10, xx2 - xx1)
        h = jnp.maximum(1e-10, yy2 - yy1)
        inter = w * h
        iou = inter / (ca + ra - inter + 1e-14)
        overf = jnp.where((iou > NMS_THRESH) & tri, 1.0, 0.0)

        def matmul(a, b):
            return jax.lax.dot_general(
                a, b, (((1,), (0,)), ((), ())),
                preferred_element_type=jnp.float32)

        def fstep(row):
            cnt = matmul(row, overf)
            return a_row * jnp.where(cnt == 0.0, 1.0, 0.0)

        def wcond(carry):
            return carry[1]

        def wbody(carry):
            row, _ = carry
            nrow = fstep(fstep(row))
            changed = jnp.sum(jnp.abs(nrow - row)) > 0.0
            return (nrow, changed)

        krow, _ = jax.lax.while_loop(
            wcond, wbody, (a_row, jnp.bool_(True)))
        keep_ref[t] = krow
        kc = trans(krow)

        excl_col = jnp.sum(ltf * krow, axis=1, keepdims=True)
        g_col = excl_col + jnp.float32(m)
        d0 = m // B

        def scatter_to(d, dst_off):
            q = jnp.where((g_col - jnp.float32(dst_off) == lane) & (kc > 0.5),
                          1.0, 0.0)
            pk_ref[0, d] = pk_ref[0, d] + jnp.sum(q * cx1, axis=0, keepdims=True)
            pk_ref[1, d] = pk_ref[1, d] + jnp.sum(q * cy1, axis=0, keepdims=True)
            pk_ref[2, d] = pk_ref[2, d] + jnp.sum(q * cx2, axis=0, keepdims=True)
            pk_ref[3, d] = pk_ref[3, d] + jnp.sum(q * cy2, axis=0, keepdims=True)

        k_cnt = jnp.sum(krow).astype(jnp.int32)
        scatter_to(d0, d0 * B)

        @pl.when(m + k_cnt > (d0 + 1) * B)
        def _():
            scatter_to(d0 + 1, (d0 + 1) * B)

        return m + k_cnt

    jax.lax.fori_loop(0, T, tile_body, jnp.int32(0))


def _run_nms(xr):
    return pl.pallas_call(
        _nms_tiles_kernel,
        out_shape=jax.ShapeDtypeStruct((T, 1, B), jnp.float32),
        scratch_shapes=[pltpu.VMEM((B, B), jnp.float32),
                        pltpu.VMEM((4, T + 1, 1, B), jnp.float32)],
    )(xr)


def kernel(boxes, scores):
    xy1 = boxes[:, :2] - boxes[:, 2:] * 0.5
    xy2 = boxes[:, :2] + boxes[:, 2:] * 0.5
    boxes_xyxy = jnp.concatenate([xy1, xy2], axis=-1)
    order = jnp.argsort(-scores)
    b_sorted = boxes_xyxy[order]
    bp = jnp.zeros((P, 4), jnp.float32).at[:N_BOXES].set(b_sorted)
    xr = bp.T.reshape(4, T, 1, B)
    keep_tiles = _run_nms(xr)
    keep_sorted = keep_tiles.reshape(P)[:N_BOXES]
    kf = jnp.zeros((N_BOXES,), jnp.float32).at[order].set(keep_sorted)
    out = jnp.concatenate(
        [boxes_xyxy * kf[:, None], (scores * kf)[:, None]], axis=-1)
    return out


# speedup vs baseline: 1.0072x; 1.0072x over previous
import jax
import jax.numpy as jnp
from jax.experimental import pallas as pl
from jax.experimental.pallas import tpu as pltpu

N_BOXES = 20000
B = 256
T = 80
P = T * B
NMS_THRESH = 0.5


def _nms_tiles_kernel(xr_ref, keep_ref, ident_ref, pk_ref):
    ident_ref[...] = (
        jax.lax.broadcasted_iota(jnp.int32, (B, B), 0)
        == jax.lax.broadcasted_iota(jnp.int32, (B, B), 1)
    ).astype(jnp.float32)
    pk_ref[...] = jnp.zeros((4, T + 1, 1, B), jnp.float32)

    tri = (
        jax.lax.broadcasted_iota(jnp.int32, (B, B), 0)
        < jax.lax.broadcasted_iota(jnp.int32, (B, B), 1)
    )
    ltf = jnp.where(
        jax.lax.broadcasted_iota(jnp.int32, (B, B), 1)
        < jax.lax.broadcasted_iota(jnp.int32, (B, B), 0), 1.0, 0.0)
    lane = jax.lax.broadcasted_iota(jnp.int32, (1, B), 1).astype(jnp.float32)

    def trans(row):
        return jnp.sum(ident_ref[...] * row, axis=1, keepdims=True)

    def trans_back(col):
        return jnp.sum(ident_ref[...] * col, axis=0, keepdims=True)

    def tile_body(t, m):
        rx1 = xr_ref[0, t]
        ry1 = xr_ref[1, t]
        rx2 = xr_ref[2, t]
        ry2 = xr_ref[3, t]
        cx1 = trans(rx1)
        cy1 = trans(ry1)
        cx2 = trans(rx2)
        cy2 = trans(ry2)
        ra = (rx2 - rx1) * (ry2 - ry1)
        ca = (cx2 - cx1) * (cy2 - cy1)

        def sweep(p, sup):
            px1 = pk_ref[0, p]
            py1 = pk_ref[1, p]
            px2 = pk_ref[2, p]
            py2 = pk_ref[3, p]
            pa = (px2 - px1) * (py2 - py1)
            xx1 = jnp.maximum(px1, cx1)
            yy1 = jnp.maximum(py1, cy1)
            xx2 = jnp.minimum(px2, cx2)
            yy2 = jnp.minimum(py2, cy2)
            w = jnp.maximum(1e-10, xx2 - xx1)
            h = jnp.maximum(1e-10, yy2 - yy1)
            inter = w * h
            iou = inter / (pa + ca - inter + 1e-14)
            s = jnp.max(jnp.where(iou > NMS_THRESH, 1.0, 0.0),
                        axis=1, keepdims=True)
            return jnp.maximum(sup, s)

        n_blocks = (m + (B - 1)) // B
        sup_col = jax.lax.fori_loop(0, n_blocks, sweep,
                                    jnp.zeros((B, 1), jnp.float32))
        a_row = trans_back(1.0 - sup_col)

        xx1 = jnp.maximum(cx1, rx1)
        yy1 = jnp.maximum(cy1, ry1)
        xx2 = jnp.minimum(cx2, rx2)
        yy2 = jnp.minimum(cy2, ry2)
        w = jnp.maximum(1e-10, xx2 - xx1)
        h = jnp.maximum(1e-10, yy2 - yy1)
        inter = w * h
        iou = inter / (ca + ra - inter + 1e-14)
        overf = jnp.where((iou > NMS_THRESH) & tri, 1.0, 0.0)

        def wcond(carry):
            return carry[2]

        def wbody(carry):
            kc, _, _ = carry
            sup = jnp.max(overf * kc, axis=0, keepdims=True)
            nrow = a_row * (1.0 - sup)
            nkc = trans(nrow)
            changed = jnp.sum(jnp.abs(nkc - kc)) > 0.0
            return (nkc, nrow, changed)

        kc, krow, _ = jax.lax.while_loop(
            wcond, wbody, (trans(a_row), a_row, jnp.bool_(True)))
        keep_ref[t] = krow

        excl_col = jnp.sum(ltf * krow, axis=1, keepdims=True)
        g_col = excl_col + jnp.float32(m)
        d0 = m // B

        def scatter_to(d, dst_off):
            q = jnp.where((g_col - jnp.float32(dst_off) == lane) & (kc > 0.5),
                          1.0, 0.0)
            pk_ref[0, d] = pk_ref[0, d] + jnp.sum(q * cx1, axis=0, keepdims=True)
            pk_ref[1, d] = pk_ref[1, d] + jnp.sum(q * cy1, axis=0, keepdims=True)
            pk_ref[2, d] = pk_ref[2, d] + jnp.sum(q * cx2, axis=0, keepdims=True)
            pk_ref[3, d] = pk_ref[3, d] + jnp.sum(q * cy2, axis=0, keepdims=True)

        scatter_to(d0, d0 * B)
        scatter_to(d0 + 1, (d0 + 1) * B)
        k_cnt = jnp.sum(krow).astype(jnp.int32)
        return m + k_cnt

    jax.lax.fori_loop(0, T, tile_body, jnp.int32(0))


def _run_nms(xr):
    return pl.pallas_call(
        _nms_tiles_kernel,
        out_shape=jax.ShapeDtypeStruct((T, 1, B), jnp.float32),
        scratch_shapes=[pltpu.VMEM((B, B), jnp.float32),
                        pltpu.VMEM((4, T + 1, 1, B), jnp.float32)],
    )(xr)


def kernel(boxes, scores):
    xy1 = boxes[:, :2] - boxes[:, 2:] * 0.5
    xy2 = boxes[:, :2] + boxes[:, 2:] * 0.5
    boxes_xyxy = jnp.concatenate([xy1, xy2], axis=-1)
    order = jnp.argsort(-scores)
    b_sorted = boxes_xyxy[order]
    bp = jnp.zeros((P, 4), jnp.float32).at[:N_BOXES].set(b_sorted)
    xr = bp.T.reshape(4, T, 1, B)
    keep_tiles = _run_nms(xr)
    keep_sorted = keep_tiles.reshape(P)[:N_BOXES]
    kf = jnp.zeros((N_BOXES,), jnp.float32).at[order].set(keep_sorted)
    out = jnp.concatenate(
        [boxes_xyxy * kf[:, None], (scores * kf)[:, None]], axis=-1)
    return out
